# baseline (speedup 1.0000x reference)
import functools

import jax
import jax.numpy as jnp
from jax import lax
from jax.experimental import pallas as pl
from jax.experimental.pallas import tpu as pltpu
from jax.experimental.pallas import tpu_sc as plsc

_H = 768
_FF = 1536
_E = 64
_T = 4096
_BT = 256
_NT = _T // _BT
_BB = 128
_P = _T + _E * _BB
_G = _P // _BB
_NW = 32
_CHUNK = _T // _NW


def _router_body(x_ref, gw_ref, eid_ref, wgt_ref, rank_ref, cnt_ref,
                 pstart_ref, te_ref, valid_ref, gx_ref):
    i = pl.program_id(0)
    x = x_ref[...]
    logits = lax.dot_general(x, gw_ref[...], (((1,), (1,)), ((), ())),
                             preferred_element_type=jnp.float32)
    lmax = jnp.max(logits, axis=1, keepdims=True)
    ex = jnp.exp(logits - lmax)
    wgt = 1.0 / jnp.sum(ex, axis=1, keepdims=True)
    eid = jnp.argmax(logits, axis=1).astype(jnp.int32)
    col = lax.broadcasted_iota(jnp.int32, (_BT, _E), 1)
    onehot = (col == eid[:, None]).astype(jnp.float32)
    r = lax.broadcasted_iota(jnp.int32, (_BT, _BT), 0)
    c = lax.broadcasted_iota(jnp.int32, (_BT, _BT), 1)
    tril = (c < r).astype(jnp.float32)
    prior = lax.dot_general(tril, onehot, (((1,), (0,)), ((), ())),
                            preferred_element_type=jnp.float32)

    @pl.when(i == 0)
    def _():
        cnt_ref[...] = jnp.zeros_like(cnt_ref)

    running = cnt_ref[...]
    rank = jnp.sum((prior + running) * onehot, axis=1, keepdims=True)
    cnt = running + jnp.sum(onehot, axis=0, keepdims=True)
    cnt_ref[...] = cnt
    eid_ref[...] = eid[:, None]
    wgt_ref[...] = wgt
    rank_ref[...] = rank.astype(jnp.int32)

    @pl.when(i == _NT - 1)
    def _():
        pc = jnp.ceil(cnt * (1.0 / _BB)) * _BB
        er = lax.broadcasted_iota(jnp.int32, (_E, _E), 0)
        ec = lax.broadcasted_iota(jnp.int32, (_E, _E), 1)
        m = (er <= ec).astype(jnp.float32)
        csum = lax.dot_general(pc, m, (((1,), (0,)), ((), ())),
                               preferred_element_type=jnp.float32)
        pstart_ref[...] = (csum - pc).astype(jnp.int32)

        total = jnp.max(csum)
        gb = (lax.broadcasted_iota(jnp.int32, (_G, 1), 0) * _BB
              ).astype(jnp.float32)
        csum_b = jnp.broadcast_to(csum, (_G, _E))
        te = jnp.sum((csum_b <= gb).astype(jnp.float32), axis=1, keepdims=True)
        last_used = jnp.sum((csum < total).astype(jnp.float32))
        te_ref[...] = jnp.minimum(te, last_used).astype(jnp.int32)
        valid_ref[...] = (gb < total).astype(jnp.int32)
        gi = lax.broadcasted_iota(jnp.int32, (_G, 1), 0).astype(jnp.float32)
        gx_ref[...] = jnp.minimum(gi, total * (1.0 / _BB) - 1.0
                                  ).astype(jnp.int32)


_NCH = 4
_CH = _CHUNK // _NCH


def _sc_scatter_body(wgt_hbm, eid_hbm, rank_hbm, ps_hbm,
                     src_hbm, ww_hbm, pos_hbm,
                     tok_v, w_v, eid_v, rank_v, base_v, i0, i1, i2, i3,
                     sem_m, sem_r, sem_w, sem_p):
    wid = lax.axis_index("s") * 2 + lax.axis_index("c")
    base = wid * _CHUNK
    idxs = (i0, i1, i2, i3)
    cp_e = pltpu.async_copy(eid_hbm.at[pl.ds(base, _CHUNK)], eid_v, sem_m)
    cp_k = pltpu.async_copy(rank_hbm.at[pl.ds(base, _CHUNK)], rank_v, sem_m)
    cp_w = pltpu.async_copy(wgt_hbm.at[pl.ds(base, _CHUNK)], w_v, sem_w)
    cp_e.wait()
    cp_g = pltpu.async_copy(ps_hbm.at[eid_v], base_v, sem_m)
    cp_k.wait()
    cp_g.wait()
    for k in range(_NCH):
        for j in range(_CH // 16):
            sl = pl.ds(k * _CH + j * 16, 16)
            idxs[k][pl.ds(j * 16, 16)] = base_v[sl] + rank_v[sl]
            tok_v[sl] = lax.iota(jnp.int32, 16) + (base + k * _CH + j * 16)
    pend = [pltpu.async_copy(idxs[k], pos_hbm.at[pl.ds(base + k * _CH, _CH)],
                             sem_p)
            for k in range(_NCH)]
    cp_w.wait()
    for k in range(_NCH):
        pend.append(pltpu.async_copy(w_v.at[pl.ds(k * _CH, _CH)],
                                     ww_hbm.at[idxs[k]], sem_w))
        pend.append(pltpu.async_copy(tok_v.at[pl.ds(k * _CH, _CH)],
                                     src_hbm.at[idxs[k]], sem_r))
    for cp in pend:
        cp.wait()


def _sc_gather_body(ys_hbm, pos_hbm, out_hbm, idx_v, rows_v, sem):
    wid = lax.axis_index("s") * 2 + lax.axis_index("c")
    base = wid * _CHUNK
    pltpu.sync_copy(pos_hbm.at[pl.ds(base, _CHUNK)], idx_v)
    pltpu.async_copy(ys_hbm.at[idx_v], rows_v, sem).wait()
    pltpu.sync_copy(rows_v, out_hbm.at[pl.ds(base, _CHUNK)])


def _moe_body(te_ref, valid_ref, gx_ref, src_ref, x_hbm, wsg_ref, wsu_ref,
              w2a_ref, w2b_ref, ww_ref, out_ref, xbuf, sem):
    g = pl.program_id(0)

    @pl.when(valid_ref[g] != 0)
    def _():
        cps = []
        for r in range(_BB):
            t = jnp.clip(src_ref[g * _BB + r], 0, _T - 1)
            cp = pltpu.make_async_copy(x_hbm.at[pl.ds(t, 1)],
                                       xbuf.at[pl.ds(r, 1)], sem)
            cp.start()
            cps.append(cp)
        for cp in cps:
            cp.wait()
        x = xbuf[...]
        gate = lax.dot_general(x, wsg_ref[0, 0], (((1,), (1,)), ((), ())),
                               preferred_element_type=jnp.float32)
        up = lax.dot_general(x, wsu_ref[0, 0], (((1,), (1,)), ((), ())),
                             preferred_element_type=jnp.float32)
        h = (gate * jax.nn.sigmoid(gate) * up) * ww_ref[...]
        ya = lax.dot_general(h, w2a_ref[0], (((1,), (1,)), ((), ())),
                             preferred_element_type=jnp.float32)
        yb = lax.dot_general(h, w2b_ref[0], (((1,), (1,)), ((), ())),
                             preferred_element_type=jnp.float32)
        out_ref[:, : _H // 2] = ya
        out_ref[:, _H // 2:] = yb


def _router_call(x, gate_w):
    return pl.pallas_call(
        _router_body,
        grid=(_NT,),
        in_specs=[
            pl.BlockSpec((_BT, _H), lambda i: (i, 0)),
            pl.BlockSpec((_E, _H), lambda i: (0, 0)),
        ],
        out_specs=[
            pl.BlockSpec((_BT, 1), lambda i: (i, 0)),
            pl.BlockSpec((_BT, 1), lambda i: (i, 0)),
            pl.BlockSpec((_BT, 1), lambda i: (i, 0)),
            pl.BlockSpec((1, _E), lambda i: (0, 0)),
            pl.BlockSpec((1, _E), lambda i: (0, 0)),
            pl.BlockSpec((_G, 1), lambda i: (0, 0)),
            pl.BlockSpec((_G, 1), lambda i: (0, 0)),
            pl.BlockSpec((_G, 1), lambda i: (0, 0)),
        ],
        out_shape=[
            jax.ShapeDtypeStruct((_T, 1), jnp.int32),
            jax.ShapeDtypeStruct((_T, 1), jnp.float32),
            jax.ShapeDtypeStruct((_T, 1), jnp.int32),
            jax.ShapeDtypeStruct((1, _E), jnp.float32),
            jax.ShapeDtypeStruct((1, _E), jnp.int32),
            jax.ShapeDtypeStruct((_G, 1), jnp.int32),
            jax.ShapeDtypeStruct((_G, 1), jnp.int32),
            jax.ShapeDtypeStruct((_G, 1), jnp.int32),
        ],
    )(x, gate_w)


def _scatter_call(wgt, eid, rank, pstart):
    f = pl.kernel(
        _sc_scatter_body,
        out_type=[
            jax.ShapeDtypeStruct((_P,), jnp.int32),
            jax.ShapeDtypeStruct((_P,), jnp.float32),
            jax.ShapeDtypeStruct((_T,), jnp.int32),
        ],
        mesh=plsc.VectorSubcoreMesh(core_axis_name="c", subcore_axis_name="s"),
        scratch_types=[
            pltpu.VMEM((_CHUNK,), jnp.int32),
            pltpu.VMEM((_CHUNK,), jnp.float32),
            pltpu.VMEM((_CHUNK,), jnp.int32),
            pltpu.VMEM((_CHUNK,), jnp.int32),
            pltpu.VMEM((_CHUNK,), jnp.int32),
            pltpu.VMEM((_CH,), jnp.int32),
            pltpu.VMEM((_CH,), jnp.int32),
            pltpu.VMEM((_CH,), jnp.int32),
            pltpu.VMEM((_CH,), jnp.int32),
            pltpu.SemaphoreType.DMA,
            pltpu.SemaphoreType.DMA,
            pltpu.SemaphoreType.DMA,
            pltpu.SemaphoreType.DMA,
        ],
    )
    return f(wgt, eid, rank, pstart)


def _gather_call(ys, pos):
    f = pl.kernel(
        _sc_gather_body,
        out_type=jax.ShapeDtypeStruct((_T, _H), jnp.float32),
        mesh=plsc.VectorSubcoreMesh(core_axis_name="c", subcore_axis_name="s"),
        scratch_types=[
            pltpu.VMEM((_CHUNK,), jnp.int32),
            pltpu.VMEM((_CHUNK, _H), jnp.float32),
            pltpu.SemaphoreType.DMA,
        ],
    )
    return f(ys, pos)


def _moe_call(te, valid, gx, src, x, ws, w2s, ww):
    ws4 = ws.reshape(_E, 2, _FF, _H)
    grid_spec = pltpu.PrefetchScalarGridSpec(
        num_scalar_prefetch=4,
        grid=(_G,),
        in_specs=[
            pl.BlockSpec(memory_space=pl.ANY),
            pl.BlockSpec((1, 1, _FF, _H),
                         lambda g, te, v, gx, s: (te[g], 0, 0, 0)),
            pl.BlockSpec((1, 1, _FF, _H),
                         lambda g, te, v, gx, s: (te[g], 1, 0, 0)),
            pl.BlockSpec((1, _H // 2, _FF),
                         lambda g, te, v, gx, s: (te[g], 0, 0)),
            pl.BlockSpec((1, _H // 2, _FF),
                         lambda g, te, v, gx, s: (te[g], 1, 0)),
            pl.BlockSpec((_BB, 1), lambda g, te, v, gx, s: (gx[g], 0)),
        ],
        out_specs=pl.BlockSpec((_BB, _H), lambda g, te, v, gx, s: (gx[g], 0)),
        scratch_shapes=[
            pltpu.VMEM((_BB, _H), jnp.float32),
            pltpu.SemaphoreType.DMA,
        ],
    )
    return pl.pallas_call(
        _moe_body,
        grid_spec=grid_spec,
        out_shape=jax.ShapeDtypeStruct((_P, _H), jnp.float32),
        compiler_params=pltpu.CompilerParams(
            vmem_limit_bytes=100 * 1024 * 1024,
        ),
    )(te, valid, gx, src, x, ws4, ws4, w2s, w2s, ww)


def kernel(hidden_states, gate_w, ws, w2s):
    x = hidden_states
    eid, wgt, rank, _, pstart, te, valid, gx = _router_call(x, gate_w)
    src, ww, pos = _scatter_call(wgt.reshape(_T), eid.reshape(_T),
                                 rank.reshape(_T), pstart.reshape(_E))
    ys = _moe_call(te.reshape(_G), valid.reshape(_G), gx.reshape(_G),
                   src, x, ws, w2s, ww.reshape(_P, 1))
    return _gather_call(ys, pos)


# speedup vs baseline: 1.3827x; 1.3827x over previous
import functools

import jax
import jax.numpy as jnp
from jax import lax
from jax.experimental import pallas as pl
from jax.experimental.pallas import tpu as pltpu
from jax.experimental.pallas import tpu_sc as plsc

_H = 768
_FF = 1536
_E = 64
_T = 4096
_BT = 256
_NT = _T // _BT
_BB = 64
_P = _T + _E * _BB
_G = _P // _BB
_NW = 32
_CHUNK = _T // _NW


def _router_body(x_ref, gw_ref, eid_ref, wgt_ref, rank_ref, cnt_ref,
                 pstart_ref, te_ref, valid_ref, gx_ref):
    i = pl.program_id(0)
    x = x_ref[...]
    logits = lax.dot_general(x, gw_ref[...], (((1,), (1,)), ((), ())),
                             preferred_element_type=jnp.float32)
    lmax = jnp.max(logits, axis=1, keepdims=True)
    ex = jnp.exp(logits - lmax)
    wgt = 1.0 / jnp.sum(ex, axis=1, keepdims=True)
    eid = jnp.argmax(logits, axis=1).astype(jnp.int32)
    col = lax.broadcasted_iota(jnp.int32, (_BT, _E), 1)
    onehot = (col == eid[:, None]).astype(jnp.float32)
    r = lax.broadcasted_iota(jnp.int32, (_BT, _BT), 0)
    c = lax.broadcasted_iota(jnp.int32, (_BT, _BT), 1)
    tril = (c < r).astype(jnp.float32)
    prior = lax.dot_general(tril, onehot, (((1,), (0,)), ((), ())),
                            preferred_element_type=jnp.float32)

    @pl.when(i == 0)
    def _():
        cnt_ref[...] = jnp.zeros_like(cnt_ref)

    running = cnt_ref[...]
    rank = jnp.sum((prior + running) * onehot, axis=1, keepdims=True)
    cnt = running + jnp.sum(onehot, axis=0, keepdims=True)
    cnt_ref[...] = cnt
    eid_ref[...] = eid[:, None]
    wgt_ref[...] = wgt
    rank_ref[...] = rank.astype(jnp.int32)

    @pl.when(i == _NT - 1)
    def _():
        pc = jnp.ceil(cnt * (1.0 / _BB)) * _BB
        er = lax.broadcasted_iota(jnp.int32, (_E, _E), 0)
        ec = lax.broadcasted_iota(jnp.int32, (_E, _E), 1)
        m = (er <= ec).astype(jnp.float32)
        csum = lax.dot_general(pc, m, (((1,), (0,)), ((), ())),
                               preferred_element_type=jnp.float32)
        pstart_ref[...] = (csum - pc).astype(jnp.int32)

        total = jnp.max(csum)
        gb = (lax.broadcasted_iota(jnp.int32, (_G, 1), 0) * _BB
              ).astype(jnp.float32)
        csum_b = jnp.broadcast_to(csum, (_G, _E))
        te = jnp.sum((csum_b <= gb).astype(jnp.float32), axis=1, keepdims=True)
        last_used = jnp.sum((csum < total).astype(jnp.float32))
        te_ref[...] = jnp.minimum(te, last_used).astype(jnp.int32)
        valid_ref[...] = (gb < total).astype(jnp.int32)
        gi = lax.broadcasted_iota(jnp.int32, (_G, 1), 0).astype(jnp.float32)
        gx_ref[...] = jnp.minimum(gi, total * (1.0 / _BB) - 1.0
                                  ).astype(jnp.int32)


_NCH = 4
_CH = _CHUNK // _NCH


def _sc_scatter_body(x_hbm, wgt_hbm, eid_hbm, rank_hbm, ps_hbm,
                     xs_hbm, ww_hbm, pos_hbm,
                     rows_v, w_v, eid_v, rank_v, base_v, i0, i1, i2, i3,
                     sem_x, sem_m, sem_r, sem_w, sem_p):
    wid = lax.axis_index("s") * 2 + lax.axis_index("c")
    base = wid * _CHUNK
    idxs = (i0, i1, i2, i3)
    cp_x = [pltpu.async_copy(x_hbm.at[pl.ds(base + k * _CH, _CH)],
                             rows_v.at[pl.ds(k * _CH, _CH)], sem_x)
            for k in range(_NCH)]
    cp_e = pltpu.async_copy(eid_hbm.at[pl.ds(base, _CHUNK)], eid_v, sem_m)
    cp_k = pltpu.async_copy(rank_hbm.at[pl.ds(base, _CHUNK)], rank_v, sem_m)
    cp_w = pltpu.async_copy(wgt_hbm.at[pl.ds(base, _CHUNK)], w_v, sem_w)
    cp_e.wait()
    cp_g = pltpu.async_copy(ps_hbm.at[eid_v], base_v, sem_m)
    cp_k.wait()
    cp_g.wait()
    for k in range(_NCH):
        for j in range(_CH // 16):
            sl = pl.ds(k * _CH + j * 16, 16)
            idxs[k][pl.ds(j * 16, 16)] = base_v[sl] + rank_v[sl]
    pend = [pltpu.async_copy(idxs[k], pos_hbm.at[pl.ds(base + k * _CH, _CH)],
                             sem_p)
            for k in range(_NCH)]
    cp_w.wait()
    for k in range(_NCH):
        pend.append(pltpu.async_copy(w_v.at[pl.ds(k * _CH, _CH)],
                                     ww_hbm.at[idxs[k]], sem_w))
        cp_x[k].wait()
        pend.append(pltpu.async_copy(rows_v.at[pl.ds(k * _CH, _CH)],
                                     xs_hbm.at[idxs[k]], sem_r))
    for cp in pend:
        cp.wait()


def _sc_gather_body(ys_hbm, pos_hbm, out_hbm, idx_v, rows_v, sem):
    wid = lax.axis_index("s") * 2 + lax.axis_index("c")
    base = wid * _CHUNK
    pltpu.sync_copy(pos_hbm.at[pl.ds(base, _CHUNK)], idx_v)
    pltpu.async_copy(ys_hbm.at[idx_v], rows_v, sem).wait()
    pltpu.sync_copy(rows_v, out_hbm.at[pl.ds(base, _CHUNK)])


def _moe_body(te_ref, valid_ref, gx_ref, xs_ref, wsg_ref, wsu_ref,
              w2a_ref, w2b_ref, ww_ref, out_ref):
    g = pl.program_id(0)

    @pl.when(valid_ref[g] != 0)
    def _():
        x = xs_ref[...]
        gate = lax.dot_general(x, wsg_ref[0, 0], (((1,), (1,)), ((), ())),
                               preferred_element_type=jnp.float32)
        up = lax.dot_general(x, wsu_ref[0, 0], (((1,), (1,)), ((), ())),
                             preferred_element_type=jnp.float32)
        h = (gate * jax.nn.sigmoid(gate) * up) * ww_ref[...]
        ya = lax.dot_general(h, w2a_ref[0], (((1,), (1,)), ((), ())),
                             preferred_element_type=jnp.float32)
        yb = lax.dot_general(h, w2b_ref[0], (((1,), (1,)), ((), ())),
                             preferred_element_type=jnp.float32)
        out_ref[:, : _H // 2] = ya
        out_ref[:, _H // 2:] = yb


def _router_call(x, gate_w):
    return pl.pallas_call(
        _router_body,
        grid=(_NT,),
        in_specs=[
            pl.BlockSpec((_BT, _H), lambda i: (i, 0)),
            pl.BlockSpec((_E, _H), lambda i: (0, 0)),
        ],
        out_specs=[
            pl.BlockSpec((_BT, 1), lambda i: (i, 0)),
            pl.BlockSpec((_BT, 1), lambda i: (i, 0)),
            pl.BlockSpec((_BT, 1), lambda i: (i, 0)),
            pl.BlockSpec((1, _E), lambda i: (0, 0)),
            pl.BlockSpec((1, _E), lambda i: (0, 0)),
            pl.BlockSpec((_G, 1), lambda i: (0, 0)),
            pl.BlockSpec((_G, 1), lambda i: (0, 0)),
            pl.BlockSpec((_G, 1), lambda i: (0, 0)),
        ],
        out_shape=[
            jax.ShapeDtypeStruct((_T, 1), jnp.int32),
            jax.ShapeDtypeStruct((_T, 1), jnp.float32),
            jax.ShapeDtypeStruct((_T, 1), jnp.int32),
            jax.ShapeDtypeStruct((1, _E), jnp.float32),
            jax.ShapeDtypeStruct((1, _E), jnp.int32),
            jax.ShapeDtypeStruct((_G, 1), jnp.int32),
            jax.ShapeDtypeStruct((_G, 1), jnp.int32),
            jax.ShapeDtypeStruct((_G, 1), jnp.int32),
        ],
    )(x, gate_w)


def _scatter_call(x, wgt, eid, rank, pstart):
    f = pl.kernel(
        _sc_scatter_body,
        out_type=[
            jax.ShapeDtypeStruct((_P, _H), jnp.float32),
            jax.ShapeDtypeStruct((_P,), jnp.float32),
            jax.ShapeDtypeStruct((_T,), jnp.int32),
        ],
        mesh=plsc.VectorSubcoreMesh(core_axis_name="c", subcore_axis_name="s"),
        scratch_types=[
            pltpu.VMEM((_CHUNK, _H), jnp.float32),
            pltpu.VMEM((_CHUNK,), jnp.float32),
            pltpu.VMEM((_CHUNK,), jnp.int32),
            pltpu.VMEM((_CHUNK,), jnp.int32),
            pltpu.VMEM((_CHUNK,), jnp.int32),
            pltpu.VMEM((_CH,), jnp.int32),
            pltpu.VMEM((_CH,), jnp.int32),
            pltpu.VMEM((_CH,), jnp.int32),
            pltpu.VMEM((_CH,), jnp.int32),
            pltpu.SemaphoreType.DMA,
            pltpu.SemaphoreType.DMA,
            pltpu.SemaphoreType.DMA,
            pltpu.SemaphoreType.DMA,
            pltpu.SemaphoreType.DMA,
        ],
    )
    return f(x, wgt, eid, rank, pstart)


def _gather_call(ys, pos):
    f = pl.kernel(
        _sc_gather_body,
        out_type=jax.ShapeDtypeStruct((_T, _H), jnp.float32),
        mesh=plsc.VectorSubcoreMesh(core_axis_name="c", subcore_axis_name="s"),
        scratch_types=[
            pltpu.VMEM((_CHUNK,), jnp.int32),
            pltpu.VMEM((_CHUNK, _H), jnp.float32),
            pltpu.SemaphoreType.DMA,
        ],
    )
    return f(ys, pos)


def _moe_call(te, valid, gx, xs, ws, w2s, ww):
    ws4 = ws.reshape(_E, 2, _FF, _H)
    grid_spec = pltpu.PrefetchScalarGridSpec(
        num_scalar_prefetch=3,
        grid=(_G,),
        in_specs=[
            pl.BlockSpec((_BB, _H), lambda g, te, v, gx: (gx[g], 0)),
            pl.BlockSpec((1, 1, _FF, _H),
                         lambda g, te, v, gx: (te[g], 0, 0, 0)),
            pl.BlockSpec((1, 1, _FF, _H),
                         lambda g, te, v, gx: (te[g], 1, 0, 0)),
            pl.BlockSpec((1, _H // 2, _FF),
                         lambda g, te, v, gx: (te[g], 0, 0)),
            pl.BlockSpec((1, _H // 2, _FF),
                         lambda g, te, v, gx: (te[g], 1, 0)),
            pl.BlockSpec((_BB, 1), lambda g, te, v, gx: (gx[g], 0)),
        ],
        out_specs=pl.BlockSpec((_BB, _H), lambda g, te, v, gx: (gx[g], 0)),
    )
    return pl.pallas_call(
        _moe_body,
        grid_spec=grid_spec,
        out_shape=jax.ShapeDtypeStruct((_P, _H), jnp.float32),
        compiler_params=pltpu.CompilerParams(
            vmem_limit_bytes=100 * 1024 * 1024,
        ),
    )(te, valid, gx, xs, ws4, ws4, w2s, w2s, ww)


def kernel(hidden_states, gate_w, ws, w2s):
    x = hidden_states
    eid, wgt, rank, _, pstart, te, valid, gx = _router_call(x, gate_w)
    xs, ww, pos = _scatter_call(x, wgt.reshape(_T), eid.reshape(_T),
                                rank.reshape(_T), pstart.reshape(_E))
    ys = _moe_call(te.reshape(_G), valid.reshape(_G), gx.reshape(_G),
                   xs, ws, w2s, ww.reshape(_P, 1))
    return _gather_call(ys, pos)


# speedup vs baseline: 1.6972x; 1.2274x over previous
import functools

import jax
import jax.numpy as jnp
from jax import lax
from jax.experimental import pallas as pl
from jax.experimental.pallas import tpu as pltpu
from jax.experimental.pallas import tpu_sc as plsc

_H = 768
_FF = 1536
_E = 64
_T = 4096
_BT = 256
_NT = _T // _BT
_BB = 256
_P = _T + _E * _BB
_G = _P // _BB
_NW = 32
_CHUNK = _T // _NW


def _router_body(x_ref, gw_ref, eid_ref, wgt_ref, rank_ref, cnt_ref,
                 pstart_ref, te_ref, valid_ref, gx_ref):
    i = pl.program_id(0)
    x = x_ref[...]
    logits = lax.dot_general(x, gw_ref[...], (((1,), (1,)), ((), ())),
                             preferred_element_type=jnp.float32)
    lmax = jnp.max(logits, axis=1, keepdims=True)
    ex = jnp.exp(logits - lmax)
    wgt = 1.0 / jnp.sum(ex, axis=1, keepdims=True)
    eid = jnp.argmax(logits, axis=1).astype(jnp.int32)
    col = lax.broadcasted_iota(jnp.int32, (_BT, _E), 1)
    onehot = (col == eid[:, None]).astype(jnp.float32)
    r = lax.broadcasted_iota(jnp.int32, (_BT, _BT), 0)
    c = lax.broadcasted_iota(jnp.int32, (_BT, _BT), 1)
    tril = (c < r).astype(jnp.float32)
    prior = lax.dot_general(tril, onehot, (((1,), (0,)), ((), ())),
                            preferred_element_type=jnp.float32)

    @pl.when(i == 0)
    def _():
        cnt_ref[...] = jnp.zeros_like(cnt_ref)

    running = cnt_ref[...]
    rank = jnp.sum((prior + running) * onehot, axis=1, keepdims=True)
    cnt = running + jnp.sum(onehot, axis=0, keepdims=True)
    cnt_ref[...] = cnt
    eid_ref[...] = eid[:, None]
    wgt_ref[...] = wgt
    rank_ref[...] = rank.astype(jnp.int32)

    @pl.when(i == _NT - 1)
    def _():
        pc = jnp.ceil(cnt * (1.0 / _BB)) * _BB
        er = lax.broadcasted_iota(jnp.int32, (_E, _E), 0)
        ec = lax.broadcasted_iota(jnp.int32, (_E, _E), 1)
        m = (er <= ec).astype(jnp.float32)
        csum = lax.dot_general(pc, m, (((1,), (0,)), ((), ())),
                               preferred_element_type=jnp.float32)
        pstart_ref[...] = (csum - pc).astype(jnp.int32)

        total = jnp.max(csum)
        gb = (lax.broadcasted_iota(jnp.int32, (_G, 1), 0) * _BB
              ).astype(jnp.float32)
        csum_b = jnp.broadcast_to(csum, (_G, _E))
        te = jnp.sum((csum_b <= gb).astype(jnp.float32), axis=1, keepdims=True)
        last_used = jnp.sum((csum < total).astype(jnp.float32))
        te_ref[...] = jnp.minimum(te, last_used).astype(jnp.int32)
        valid_ref[...] = (gb < total).astype(jnp.int32)
        gi = lax.broadcasted_iota(jnp.int32, (_G, 1), 0).astype(jnp.float32)
        gx_ref[...] = jnp.minimum(gi, total * (1.0 / _BB) - 1.0
                                  ).astype(jnp.int32)


_NCH = 4
_CH = _CHUNK // _NCH


def _sc_scatter_body(x_hbm, wgt_hbm, eid_hbm, rank_hbm, ps_hbm,
                     xs_hbm, ww_hbm, pos_hbm,
                     rows_v, w_v, eid_v, rank_v, base_v, i0, i1, i2, i3,
                     sem_x, sem_m, sem_r, sem_w, sem_p):
    wid = lax.axis_index("s") * 2 + lax.axis_index("c")
    base = wid * _CHUNK
    idxs = (i0, i1, i2, i3)
    cp_x = [pltpu.async_copy(x_hbm.at[pl.ds(base + k * _CH, _CH)],
                             rows_v.at[pl.ds(k * _CH, _CH)], sem_x)
            for k in range(_NCH)]
    cp_e = pltpu.async_copy(eid_hbm.at[pl.ds(base, _CHUNK)], eid_v, sem_m)
    cp_k = pltpu.async_copy(rank_hbm.at[pl.ds(base, _CHUNK)], rank_v, sem_m)
    cp_w = pltpu.async_copy(wgt_hbm.at[pl.ds(base, _CHUNK)], w_v, sem_w)
    cp_e.wait()
    cp_g = pltpu.async_copy(ps_hbm.at[eid_v], base_v, sem_m)
    cp_k.wait()
    cp_g.wait()
    for k in range(_NCH):
        for j in range(_CH // 16):
            sl = pl.ds(k * _CH + j * 16, 16)
            idxs[k][pl.ds(j * 16, 16)] = base_v[sl] + rank_v[sl]
    pend = [pltpu.async_copy(idxs[k], pos_hbm.at[pl.ds(base + k * _CH, _CH)],
                             sem_p)
            for k in range(_NCH)]
    cp_w.wait()
    for k in range(_NCH):
        pend.append(pltpu.async_copy(w_v.at[pl.ds(k * _CH, _CH)],
                                     ww_hbm.at[idxs[k]], sem_w))
        cp_x[k].wait()
        pend.append(pltpu.async_copy(rows_v.at[pl.ds(k * _CH, _CH)],
                                     xs_hbm.at[idxs[k]], sem_r))
    for cp in pend:
        cp.wait()


def _sc_gather_body(ys_hbm, pos_hbm, out_hbm, idx_v, rows_v, sem):
    wid = lax.axis_index("s") * 2 + lax.axis_index("c")
    base = wid * _CHUNK
    pltpu.sync_copy(pos_hbm.at[pl.ds(base, _CHUNK)], idx_v)
    pltpu.async_copy(ys_hbm.at[idx_v], rows_v, sem).wait()
    pltpu.sync_copy(rows_v, out_hbm.at[pl.ds(base, _CHUNK)])


def _moe_body(te_ref, valid_ref, gx_ref, xs_ref, wsg_ref, wsu_ref,
              w2a_ref, w2b_ref, ww_ref, out_ref):
    g = pl.program_id(0)

    @pl.when(valid_ref[g] != 0)
    def _():
        x = xs_ref[...]
        gate = lax.dot_general(x, wsg_ref[0, 0], (((1,), (1,)), ((), ())),
                               preferred_element_type=jnp.float32)
        up = lax.dot_general(x, wsu_ref[0, 0], (((1,), (1,)), ((), ())),
                             preferred_element_type=jnp.float32)
        h = (gate * jax.nn.sigmoid(gate) * up) * ww_ref[...]
        ya = lax.dot_general(h, w2a_ref[0], (((1,), (1,)), ((), ())),
                             preferred_element_type=jnp.float32)
        yb = lax.dot_general(h, w2b_ref[0], (((1,), (1,)), ((), ())),
                             preferred_element_type=jnp.float32)
        out_ref[:, : _H // 2] = ya
        out_ref[:, _H // 2:] = yb


def _router_call(x, gate_w):
    return pl.pallas_call(
        _router_body,
        grid=(_NT,),
        in_specs=[
            pl.BlockSpec((_BT, _H), lambda i: (i, 0)),
            pl.BlockSpec((_E, _H), lambda i: (0, 0)),
        ],
        out_specs=[
            pl.BlockSpec((_BT, 1), lambda i: (i, 0)),
            pl.BlockSpec((_BT, 1), lambda i: (i, 0)),
            pl.BlockSpec((_BT, 1), lambda i: (i, 0)),
            pl.BlockSpec((1, _E), lambda i: (0, 0)),
            pl.BlockSpec((1, _E), lambda i: (0, 0)),
            pl.BlockSpec((_G, 1), lambda i: (0, 0)),
            pl.BlockSpec((_G, 1), lambda i: (0, 0)),
            pl.BlockSpec((_G, 1), lambda i: (0, 0)),
        ],
        out_shape=[
            jax.ShapeDtypeStruct((_T, 1), jnp.int32),
            jax.ShapeDtypeStruct((_T, 1), jnp.float32),
            jax.ShapeDtypeStruct((_T, 1), jnp.int32),
            jax.ShapeDtypeStruct((1, _E), jnp.float32),
            jax.ShapeDtypeStruct((1, _E), jnp.int32),
            jax.ShapeDtypeStruct((_G, 1), jnp.int32),
            jax.ShapeDtypeStruct((_G, 1), jnp.int32),
            jax.ShapeDtypeStruct((_G, 1), jnp.int32),
        ],
    )(x, gate_w)


def _scatter_call(x, wgt, eid, rank, pstart):
    f = pl.kernel(
        _sc_scatter_body,
        out_type=[
            jax.ShapeDtypeStruct((_P, _H), jnp.float32),
            jax.ShapeDtypeStruct((_P,), jnp.float32),
            jax.ShapeDtypeStruct((_T,), jnp.int32),
        ],
        mesh=plsc.VectorSubcoreMesh(core_axis_name="c", subcore_axis_name="s"),
        scratch_types=[
            pltpu.VMEM((_CHUNK, _H), jnp.float32),
            pltpu.VMEM((_CHUNK,), jnp.float32),
            pltpu.VMEM((_CHUNK,), jnp.int32),
            pltpu.VMEM((_CHUNK,), jnp.int32),
            pltpu.VMEM((_CHUNK,), jnp.int32),
            pltpu.VMEM((_CH,), jnp.int32),
            pltpu.VMEM((_CH,), jnp.int32),
            pltpu.VMEM((_CH,), jnp.int32),
            pltpu.VMEM((_CH,), jnp.int32),
            pltpu.SemaphoreType.DMA,
            pltpu.SemaphoreType.DMA,
            pltpu.SemaphoreType.DMA,
            pltpu.SemaphoreType.DMA,
            pltpu.SemaphoreType.DMA,
        ],
    )
    return f(x, wgt, eid, rank, pstart)


def _gather_call(ys, pos):
    f = pl.kernel(
        _sc_gather_body,
        out_type=jax.ShapeDtypeStruct((_T, _H), jnp.float32),
        mesh=plsc.VectorSubcoreMesh(core_axis_name="c", subcore_axis_name="s"),
        scratch_types=[
            pltpu.VMEM((_CHUNK,), jnp.int32),
            pltpu.VMEM((_CHUNK, _H), jnp.float32),
            pltpu.SemaphoreType.DMA,
        ],
    )
    return f(ys, pos)


def _moe_call(te, valid, gx, xs, ws, w2s, ww):
    ws4 = ws.reshape(_E, 2, _FF, _H)
    grid_spec = pltpu.PrefetchScalarGridSpec(
        num_scalar_prefetch=3,
        grid=(_G,),
        in_specs=[
            pl.BlockSpec((_BB, _H), lambda g, te, v, gx: (gx[g], 0)),
            pl.BlockSpec((1, 1, _FF, _H),
                         lambda g, te, v, gx: (te[g], 0, 0, 0)),
            pl.BlockSpec((1, 1, _FF, _H),
                         lambda g, te, v, gx: (te[g], 1, 0, 0)),
            pl.BlockSpec((1, _H // 2, _FF),
                         lambda g, te, v, gx: (te[g], 0, 0)),
            pl.BlockSpec((1, _H // 2, _FF),
                         lambda g, te, v, gx: (te[g], 1, 0)),
            pl.BlockSpec((_BB, 1), lambda g, te, v, gx: (gx[g], 0)),
        ],
        out_specs=pl.BlockSpec((_BB, _H), lambda g, te, v, gx: (gx[g], 0)),
    )
    return pl.pallas_call(
        _moe_body,
        grid_spec=grid_spec,
        out_shape=jax.ShapeDtypeStruct((_P, _H), jnp.float32),
        compiler_params=pltpu.CompilerParams(
            vmem_limit_bytes=100 * 1024 * 1024,
        ),
    )(te, valid, gx, xs, ws4, ws4, w2s, w2s, ww)


def kernel(hidden_states, gate_w, ws, w2s):
    x = hidden_states
    eid, wgt, rank, _, pstart, te, valid, gx = _router_call(x, gate_w)
    xs, ww, pos = _scatter_call(x, wgt.reshape(_T), eid.reshape(_T),
                                rank.reshape(_T), pstart.reshape(_E))
    ys = _moe_call(te.reshape(_G), valid.reshape(_G), gx.reshape(_G),
                   xs, ws, w2s, ww.reshape(_P, 1))
    return _gather_call(ys, pos)


# speedup vs baseline: 1.7728x; 1.0445x over previous
import functools

import jax
import jax.numpy as jnp
from jax import lax
from jax.experimental import pallas as pl
from jax.experimental.pallas import tpu as pltpu
from jax.experimental.pallas import tpu_sc as plsc

_H = 768
_FF = 1536
_E = 64
_T = 4096
_BT = 256
_NT = _T // _BT
_BB = 128
_P = _T + _E * _BB
_G = _P // _BB
_NW = 32
_CHUNK = _T // _NW


def _router_body(x_ref, gw_ref, eid_ref, wgt_ref, rank_ref, cnt_ref,
                 pstart_ref, te_ref, valid_ref, gx_ref):
    i = pl.program_id(0)
    x = x_ref[...]
    logits = lax.dot_general(x, gw_ref[...], (((1,), (1,)), ((), ())),
                             preferred_element_type=jnp.float32)
    lmax = jnp.max(logits, axis=1, keepdims=True)
    ex = jnp.exp(logits - lmax)
    wgt = 1.0 / jnp.sum(ex, axis=1, keepdims=True)
    eid = jnp.argmax(logits, axis=1).astype(jnp.int32)
    col = lax.broadcasted_iota(jnp.int32, (_BT, _E), 1)
    onehot = (col == eid[:, None]).astype(jnp.float32)
    r = lax.broadcasted_iota(jnp.int32, (_BT, _BT), 0)
    c = lax.broadcasted_iota(jnp.int32, (_BT, _BT), 1)
    tril = (c < r).astype(jnp.float32)
    prior = lax.dot_general(tril, onehot, (((1,), (0,)), ((), ())),
                            preferred_element_type=jnp.float32)

    @pl.when(i == 0)
    def _():
        cnt_ref[...] = jnp.zeros_like(cnt_ref)

    running = cnt_ref[...]
    rank = jnp.sum((prior + running) * onehot, axis=1, keepdims=True)
    cnt = running + jnp.sum(onehot, axis=0, keepdims=True)
    cnt_ref[...] = cnt
    eid_ref[...] = eid[:, None]
    wgt_ref[...] = wgt
    rank_ref[...] = rank.astype(jnp.int32)

    @pl.when(i == _NT - 1)
    def _():
        pc = jnp.ceil(cnt * (1.0 / _BB)) * _BB
        er = lax.broadcasted_iota(jnp.int32, (_E, _E), 0)
        ec = lax.broadcasted_iota(jnp.int32, (_E, _E), 1)
        m = (er <= ec).astype(jnp.float32)
        csum = lax.dot_general(pc, m, (((1,), (0,)), ((), ())),
                               preferred_element_type=jnp.float32)
        pstart_ref[...] = (csum - pc).astype(jnp.int32)

        total = jnp.max(csum)
        gb = (lax.broadcasted_iota(jnp.int32, (_G, 1), 0) * _BB
              ).astype(jnp.float32)
        csum_b = jnp.broadcast_to(csum, (_G, _E))
        te = jnp.sum((csum_b <= gb).astype(jnp.float32), axis=1, keepdims=True)
        last_used = jnp.sum((csum < total).astype(jnp.float32))
        te_ref[...] = jnp.minimum(te, last_used).astype(jnp.int32)
        valid_ref[...] = (gb < total).astype(jnp.int32)
        gi = lax.broadcasted_iota(jnp.int32, (_G, 1), 0).astype(jnp.float32)
        gx_ref[...] = jnp.minimum(gi, total * (1.0 / _BB) - 1.0
                                  ).astype(jnp.int32)


_NCH = 4
_CH = _CHUNK // _NCH


def _sc_scatter_body(x_hbm, wgt_hbm, eid_hbm, rank_hbm, ps_hbm,
                     xs_hbm, ww_hbm, pos_hbm,
                     rows_v, w_v, eid_v, rank_v, base_v, i0, i1, i2, i3,
                     sem_x, sem_m, sem_r, sem_w, sem_p):
    wid = lax.axis_index("s") * 2 + lax.axis_index("c")
    base = wid * _CHUNK
    idxs = (i0, i1, i2, i3)
    cp_x = [pltpu.async_copy(x_hbm.at[pl.ds(base + k * _CH, _CH)],
                             rows_v.at[pl.ds(k * _CH, _CH)], sem_x)
            for k in range(_NCH)]
    cp_e = pltpu.async_copy(eid_hbm.at[pl.ds(base, _CHUNK)], eid_v, sem_m)
    cp_k = pltpu.async_copy(rank_hbm.at[pl.ds(base, _CHUNK)], rank_v, sem_m)
    cp_w = pltpu.async_copy(wgt_hbm.at[pl.ds(base, _CHUNK)], w_v, sem_w)
    cp_e.wait()
    cp_g = pltpu.async_copy(ps_hbm.at[eid_v], base_v, sem_m)
    cp_k.wait()
    cp_g.wait()
    for k in range(_NCH):
        for j in range(_CH // 16):
            sl = pl.ds(k * _CH + j * 16, 16)
            idxs[k][pl.ds(j * 16, 16)] = base_v[sl] + rank_v[sl]
    pend = [pltpu.async_copy(idxs[k], pos_hbm.at[pl.ds(base + k * _CH, _CH)],
                             sem_p)
            for k in range(_NCH)]
    cp_w.wait()
    for k in range(_NCH):
        pend.append(pltpu.async_copy(w_v.at[pl.ds(k * _CH, _CH)],
                                     ww_hbm.at[idxs[k]], sem_w))
        cp_x[k].wait()
        pend.append(pltpu.async_copy(rows_v.at[pl.ds(k * _CH, _CH)],
                                     xs_hbm.at[idxs[k]], sem_r))
    for cp in pend:
        cp.wait()


def _sc_gather_body(ys_hbm, pos_hbm, out_hbm, idx_v, rows_v, sem):
    wid = lax.axis_index("s") * 2 + lax.axis_index("c")
    base = wid * _CHUNK
    pltpu.sync_copy(pos_hbm.at[pl.ds(base, _CHUNK)], idx_v)
    pltpu.async_copy(ys_hbm.at[idx_v], rows_v, sem).wait()
    pltpu.sync_copy(rows_v, out_hbm.at[pl.ds(base, _CHUNK)])


def _moe_body(te_ref, valid_ref, gx_ref, xs_ref, wsg_ref, wsu_ref,
              w2a_ref, w2b_ref, ww_ref, out_ref):
    g = pl.program_id(0)

    @pl.when(valid_ref[g] != 0)
    def _():
        x = xs_ref[...]
        gate = lax.dot_general(x, wsg_ref[0, 0], (((1,), (1,)), ((), ())),
                               preferred_element_type=jnp.float32)
        up = lax.dot_general(x, wsu_ref[0, 0], (((1,), (1,)), ((), ())),
                             preferred_element_type=jnp.float32)
        h = (gate * jax.nn.sigmoid(gate) * up) * ww_ref[...]
        ya = lax.dot_general(h, w2a_ref[0], (((1,), (1,)), ((), ())),
                             preferred_element_type=jnp.float32)
        yb = lax.dot_general(h, w2b_ref[0], (((1,), (1,)), ((), ())),
                             preferred_element_type=jnp.float32)
        out_ref[:, : _H // 2] = ya
        out_ref[:, _H // 2:] = yb


def _router_call(x, gate_w):
    return pl.pallas_call(
        _router_body,
        grid=(_NT,),
        in_specs=[
            pl.BlockSpec((_BT, _H), lambda i: (i, 0)),
            pl.BlockSpec((_E, _H), lambda i: (0, 0)),
        ],
        out_specs=[
            pl.BlockSpec((_BT, 1), lambda i: (i, 0)),
            pl.BlockSpec((_BT, 1), lambda i: (i, 0)),
            pl.BlockSpec((_BT, 1), lambda i: (i, 0)),
            pl.BlockSpec((1, _E), lambda i: (0, 0)),
            pl.BlockSpec((1, _E), lambda i: (0, 0)),
            pl.BlockSpec((_G, 1), lambda i: (0, 0)),
            pl.BlockSpec((_G, 1), lambda i: (0, 0)),
            pl.BlockSpec((_G, 1), lambda i: (0, 0)),
        ],
        out_shape=[
            jax.ShapeDtypeStruct((_T, 1), jnp.int32),
            jax.ShapeDtypeStruct((_T, 1), jnp.float32),
            jax.ShapeDtypeStruct((_T, 1), jnp.int32),
            jax.ShapeDtypeStruct((1, _E), jnp.float32),
            jax.ShapeDtypeStruct((1, _E), jnp.int32),
            jax.ShapeDtypeStruct((_G, 1), jnp.int32),
            jax.ShapeDtypeStruct((_G, 1), jnp.int32),
            jax.ShapeDtypeStruct((_G, 1), jnp.int32),
        ],
    )(x, gate_w)


def _scatter_call(x, wgt, eid, rank, pstart):
    f = pl.kernel(
        _sc_scatter_body,
        out_type=[
            jax.ShapeDtypeStruct((_P, _H), jnp.float32),
            jax.ShapeDtypeStruct((_P,), jnp.float32),
            jax.ShapeDtypeStruct((_T,), jnp.int32),
        ],
        mesh=plsc.VectorSubcoreMesh(core_axis_name="c", subcore_axis_name="s"),
        scratch_types=[
            pltpu.VMEM((_CHUNK, _H), jnp.float32),
            pltpu.VMEM((_CHUNK,), jnp.float32),
            pltpu.VMEM((_CHUNK,), jnp.int32),
            pltpu.VMEM((_CHUNK,), jnp.int32),
            pltpu.VMEM((_CHUNK,), jnp.int32),
            pltpu.VMEM((_CH,), jnp.int32),
            pltpu.VMEM((_CH,), jnp.int32),
            pltpu.VMEM((_CH,), jnp.int32),
            pltpu.VMEM((_CH,), jnp.int32),
            pltpu.SemaphoreType.DMA,
            pltpu.SemaphoreType.DMA,
            pltpu.SemaphoreType.DMA,
            pltpu.SemaphoreType.DMA,
            pltpu.SemaphoreType.DMA,
        ],
    )
    return f(x, wgt, eid, rank, pstart)


def _gather_call(ys, pos):
    f = pl.kernel(
        _sc_gather_body,
        out_type=jax.ShapeDtypeStruct((_T, _H), jnp.float32),
        mesh=plsc.VectorSubcoreMesh(core_axis_name="c", subcore_axis_name="s"),
        scratch_types=[
            pltpu.VMEM((_CHUNK,), jnp.int32),
            pltpu.VMEM((_CHUNK, _H), jnp.float32),
            pltpu.SemaphoreType.DMA,
        ],
    )
    return f(ys, pos)


def _moe_call(te, valid, gx, xs, ws, w2s, ww):
    ws4 = ws.reshape(_E, 2, _FF, _H)
    grid_spec = pltpu.PrefetchScalarGridSpec(
        num_scalar_prefetch=3,
        grid=(_G,),
        in_specs=[
            pl.BlockSpec((_BB, _H), lambda g, te, v, gx: (gx[g], 0)),
            pl.BlockSpec((1, 1, _FF, _H),
                         lambda g, te, v, gx: (te[g], 0, 0, 0)),
            pl.BlockSpec((1, 1, _FF, _H),
                         lambda g, te, v, gx: (te[g], 1, 0, 0)),
            pl.BlockSpec((1, _H // 2, _FF),
                         lambda g, te, v, gx: (te[g], 0, 0)),
            pl.BlockSpec((1, _H // 2, _FF),
                         lambda g, te, v, gx: (te[g], 1, 0)),
            pl.BlockSpec((_BB, 1), lambda g, te, v, gx: (gx[g], 0)),
        ],
        out_specs=pl.BlockSpec((_BB, _H), lambda g, te, v, gx: (gx[g], 0)),
    )
    return pl.pallas_call(
        _moe_body,
        grid_spec=grid_spec,
        out_shape=jax.ShapeDtypeStruct((_P, _H), jnp.float32),
        compiler_params=pltpu.CompilerParams(
            vmem_limit_bytes=100 * 1024 * 1024,
        ),
    )(te, valid, gx, xs, ws4, ws4, w2s, w2s, ww)


def kernel(hidden_states, gate_w, ws, w2s):
    x = hidden_states
    eid, wgt, rank, _, pstart, te, valid, gx = _router_call(x, gate_w)
    xs, ww, pos = _scatter_call(x, wgt.reshape(_T), eid.reshape(_T),
                                rank.reshape(_T), pstart.reshape(_E))
    ys = _moe_call(te.reshape(_G), valid.reshape(_G), gx.reshape(_G),
                   xs, ws, w2s, ww.reshape(_P, 1))
    return _gather_call(ys, pos)


# speedup vs baseline: 1.7937x; 1.0118x over previous
import functools

import jax
import jax.numpy as jnp
from jax import lax
from jax.experimental import pallas as pl
from jax.experimental.pallas import tpu as pltpu
from jax.experimental.pallas import tpu_sc as plsc

_H = 768
_FF = 1536
_E = 64
_T = 4096
_BT = 512
_NT = _T // _BT
_BB = 128
_P = _T + _E * _BB
_G = _P // _BB
_NW = 32
_CHUNK = _T // _NW


def _router_body(x_ref, gw_ref, eid_ref, wgt_ref, rank_ref, cnt_ref,
                 pstart_ref, te_ref, valid_ref, gx_ref):
    i = pl.program_id(0)
    x = x_ref[...]
    logits = lax.dot_general(x, gw_ref[...], (((1,), (1,)), ((), ())),
                             preferred_element_type=jnp.float32)
    lmax = jnp.max(logits, axis=1, keepdims=True)
    ex = jnp.exp(logits - lmax)
    wgt = 1.0 / jnp.sum(ex, axis=1, keepdims=True)
    eid = jnp.argmax(logits, axis=1).astype(jnp.int32)
    col = lax.broadcasted_iota(jnp.int32, (_BT, _E), 1)
    onehot = (col == eid[:, None]).astype(jnp.float32)
    r = lax.broadcasted_iota(jnp.int32, (_BT, _BT), 0)
    c = lax.broadcasted_iota(jnp.int32, (_BT, _BT), 1)
    tril = (c < r).astype(jnp.float32)
    prior = lax.dot_general(tril, onehot, (((1,), (0,)), ((), ())),
                            preferred_element_type=jnp.float32)

    @pl.when(i == 0)
    def _():
        cnt_ref[...] = jnp.zeros_like(cnt_ref)

    running = cnt_ref[...]
    rank = jnp.sum((prior + running) * onehot, axis=1, keepdims=True)
    cnt = running + jnp.sum(onehot, axis=0, keepdims=True)
    cnt_ref[...] = cnt
    eid_ref[...] = eid[:, None]
    wgt_ref[...] = wgt
    rank_ref[...] = rank.astype(jnp.int32)

    @pl.when(i == _NT - 1)
    def _():
        pc = jnp.ceil(cnt * (1.0 / _BB)) * _BB
        er = lax.broadcasted_iota(jnp.int32, (_E, _E), 0)
        ec = lax.broadcasted_iota(jnp.int32, (_E, _E), 1)
        m = (er <= ec).astype(jnp.float32)
        csum = lax.dot_general(pc, m, (((1,), (0,)), ((), ())),
                               preferred_element_type=jnp.float32)
        pstart_ref[...] = (csum - pc).astype(jnp.int32)

        total = jnp.max(csum)
        gb = (lax.broadcasted_iota(jnp.int32, (_G, 1), 0) * _BB
              ).astype(jnp.float32)
        csum_b = jnp.broadcast_to(csum, (_G, _E))
        te = jnp.sum((csum_b <= gb).astype(jnp.float32), axis=1, keepdims=True)
        last_used = jnp.sum((csum < total).astype(jnp.float32))
        te_ref[...] = jnp.minimum(te, last_used).astype(jnp.int32)
        valid_ref[...] = (gb < total).astype(jnp.int32)
        gi = lax.broadcasted_iota(jnp.int32, (_G, 1), 0).astype(jnp.float32)
        gx_ref[...] = jnp.minimum(gi, total * (1.0 / _BB) - 1.0
                                  ).astype(jnp.int32)


_NCH = 4
_CH = _CHUNK // _NCH


def _sc_scatter_body(x_hbm, wgt_hbm, eid_hbm, rank_hbm, ps_hbm,
                     xs_hbm, ww_hbm, pos_hbm,
                     rows_v, w_v, eid_v, rank_v, base_v, i0, i1, i2, i3,
                     sem_x, sem_m, sem_r, sem_w, sem_p):
    wid = lax.axis_index("s") * 2 + lax.axis_index("c")
    base = wid * _CHUNK
    idxs = (i0, i1, i2, i3)
    cp_x = [pltpu.async_copy(x_hbm.at[pl.ds(base + k * _CH, _CH)],
                             rows_v.at[pl.ds(k * _CH, _CH)], sem_x)
            for k in range(_NCH)]
    cp_e = pltpu.async_copy(eid_hbm.at[pl.ds(base, _CHUNK)], eid_v, sem_m)
    cp_k = pltpu.async_copy(rank_hbm.at[pl.ds(base, _CHUNK)], rank_v, sem_m)
    cp_w = pltpu.async_copy(wgt_hbm.at[pl.ds(base, _CHUNK)], w_v, sem_w)
    cp_e.wait()
    cp_g = pltpu.async_copy(ps_hbm.at[eid_v], base_v, sem_m)
    cp_k.wait()
    cp_g.wait()
    for k in range(_NCH):
        for j in range(_CH // 16):
            sl = pl.ds(k * _CH + j * 16, 16)
            idxs[k][pl.ds(j * 16, 16)] = base_v[sl] + rank_v[sl]
    pend = [pltpu.async_copy(idxs[k], pos_hbm.at[pl.ds(base + k * _CH, _CH)],
                             sem_p)
            for k in range(_NCH)]
    cp_w.wait()
    for k in range(_NCH):
        pend.append(pltpu.async_copy(w_v.at[pl.ds(k * _CH, _CH)],
                                     ww_hbm.at[idxs[k]], sem_w))
        cp_x[k].wait()
        pend.append(pltpu.async_copy(rows_v.at[pl.ds(k * _CH, _CH)],
                                     xs_hbm.at[idxs[k]], sem_r))
    for cp in pend:
        cp.wait()


def _sc_gather_body(ys_hbm, pos_hbm, out_hbm, idx_v, rows_v, sem):
    wid = lax.axis_index("s") * 2 + lax.axis_index("c")
    base = wid * _CHUNK
    pltpu.sync_copy(pos_hbm.at[pl.ds(base, _CHUNK)], idx_v)
    pltpu.async_copy(ys_hbm.at[idx_v], rows_v, sem).wait()
    pltpu.sync_copy(rows_v, out_hbm.at[pl.ds(base, _CHUNK)])


def _moe_body(te_ref, valid_ref, gx_ref, xs_ref, wsg_ref, wsu_ref,
              w2a_ref, w2b_ref, ww_ref, out_ref):
    g = pl.program_id(0)

    @pl.when(valid_ref[g] != 0)
    def _():
        x = xs_ref[...]
        gate = lax.dot_general(x, wsg_ref[0, 0], (((1,), (1,)), ((), ())),
                               preferred_element_type=jnp.float32)
        up = lax.dot_general(x, wsu_ref[0, 0], (((1,), (1,)), ((), ())),
                             preferred_element_type=jnp.float32)
        h = (gate * jax.nn.sigmoid(gate) * up) * ww_ref[...]
        ya = lax.dot_general(h, w2a_ref[0], (((1,), (1,)), ((), ())),
                             preferred_element_type=jnp.float32)
        yb = lax.dot_general(h, w2b_ref[0], (((1,), (1,)), ((), ())),
                             preferred_element_type=jnp.float32)
        out_ref[:, : _H // 2] = ya
        out_ref[:, _H // 2:] = yb


def _router_call(x, gate_w):
    return pl.pallas_call(
        _router_body,
        grid=(_NT,),
        in_specs=[
            pl.BlockSpec((_BT, _H), lambda i: (i, 0)),
            pl.BlockSpec((_E, _H), lambda i: (0, 0)),
        ],
        out_specs=[
            pl.BlockSpec((_BT, 1), lambda i: (i, 0)),
            pl.BlockSpec((_BT, 1), lambda i: (i, 0)),
            pl.BlockSpec((_BT, 1), lambda i: (i, 0)),
            pl.BlockSpec((1, _E), lambda i: (0, 0)),
            pl.BlockSpec((1, _E), lambda i: (0, 0)),
            pl.BlockSpec((_G, 1), lambda i: (0, 0)),
            pl.BlockSpec((_G, 1), lambda i: (0, 0)),
            pl.BlockSpec((_G, 1), lambda i: (0, 0)),
        ],
        out_shape=[
            jax.ShapeDtypeStruct((_T, 1), jnp.int32),
            jax.ShapeDtypeStruct((_T, 1), jnp.float32),
            jax.ShapeDtypeStruct((_T, 1), jnp.int32),
            jax.ShapeDtypeStruct((1, _E), jnp.float32),
            jax.ShapeDtypeStruct((1, _E), jnp.int32),
            jax.ShapeDtypeStruct((_G, 1), jnp.int32),
            jax.ShapeDtypeStruct((_G, 1), jnp.int32),
            jax.ShapeDtypeStruct((_G, 1), jnp.int32),
        ],
    )(x, gate_w)


def _scatter_call(x, wgt, eid, rank, pstart):
    f = pl.kernel(
        _sc_scatter_body,
        out_type=[
            jax.ShapeDtypeStruct((_P, _H), jnp.float32),
            jax.ShapeDtypeStruct((_P,), jnp.float32),
            jax.ShapeDtypeStruct((_T,), jnp.int32),
        ],
        mesh=plsc.VectorSubcoreMesh(core_axis_name="c", subcore_axis_name="s"),
        scratch_types=[
            pltpu.VMEM((_CHUNK, _H), jnp.float32),
            pltpu.VMEM((_CHUNK,), jnp.float32),
            pltpu.VMEM((_CHUNK,), jnp.int32),
            pltpu.VMEM((_CHUNK,), jnp.int32),
            pltpu.VMEM((_CHUNK,), jnp.int32),
            pltpu.VMEM((_CH,), jnp.int32),
            pltpu.VMEM((_CH,), jnp.int32),
            pltpu.VMEM((_CH,), jnp.int32),
            pltpu.VMEM((_CH,), jnp.int32),
            pltpu.SemaphoreType.DMA,
            pltpu.SemaphoreType.DMA,
            pltpu.SemaphoreType.DMA,
            pltpu.SemaphoreType.DMA,
            pltpu.SemaphoreType.DMA,
        ],
    )
    return f(x, wgt, eid, rank, pstart)


def _gather_call(ys, pos):
    f = pl.kernel(
        _sc_gather_body,
        out_type=jax.ShapeDtypeStruct((_T, _H), jnp.float32),
        mesh=plsc.VectorSubcoreMesh(core_axis_name="c", subcore_axis_name="s"),
        scratch_types=[
            pltpu.VMEM((_CHUNK,), jnp.int32),
            pltpu.VMEM((_CHUNK, _H), jnp.float32),
            pltpu.SemaphoreType.DMA,
        ],
    )
    return f(ys, pos)


def _moe_call(te, valid, gx, xs, ws, w2s, ww):
    ws4 = ws.reshape(_E, 2, _FF, _H)
    grid_spec = pltpu.PrefetchScalarGridSpec(
        num_scalar_prefetch=3,
        grid=(_G,),
        in_specs=[
            pl.BlockSpec((_BB, _H), lambda g, te, v, gx: (gx[g], 0)),
            pl.BlockSpec((1, 1, _FF, _H),
                         lambda g, te, v, gx: (te[g], 0, 0, 0)),
            pl.BlockSpec((1, 1, _FF, _H),
                         lambda g, te, v, gx: (te[g], 1, 0, 0)),
            pl.BlockSpec((1, _H // 2, _FF),
                         lambda g, te, v, gx: (te[g], 0, 0)),
            pl.BlockSpec((1, _H // 2, _FF),
                         lambda g, te, v, gx: (te[g], 1, 0)),
            pl.BlockSpec((_BB, 1), lambda g, te, v, gx: (gx[g], 0)),
        ],
        out_specs=pl.BlockSpec((_BB, _H), lambda g, te, v, gx: (gx[g], 0)),
    )
    return pl.pallas_call(
        _moe_body,
        grid_spec=grid_spec,
        out_shape=jax.ShapeDtypeStruct((_P, _H), jnp.float32),
        compiler_params=pltpu.CompilerParams(
            vmem_limit_bytes=100 * 1024 * 1024,
        ),
    )(te, valid, gx, xs, ws4, ws4, w2s, w2s, ww)


def kernel(hidden_states, gate_w, ws, w2s):
    x = hidden_states
    eid, wgt, rank, _, pstart, te, valid, gx = _router_call(x, gate_w)
    xs, ww, pos = _scatter_call(x, wgt.reshape(_T), eid.reshape(_T),
                                rank.reshape(_T), pstart.reshape(_E))
    ys = _moe_call(te.reshape(_G), valid.reshape(_G), gx.reshape(_G),
                   xs, ws, w2s, ww.reshape(_P, 1))
    return _gather_call(ys, pos)


# speedup vs baseline: 1.7950x; 1.0007x over previous
import functools

import jax
import jax.numpy as jnp
from jax import lax
from jax.experimental import pallas as pl
from jax.experimental.pallas import tpu as pltpu
from jax.experimental.pallas import tpu_sc as plsc

_H = 768
_FF = 1536
_E = 64
_T = 4096
_BT = 1024
_NT = _T // _BT
_BB = 128
_P = _T + _E * _BB
_G = _P // _BB
_NW = 32
_CHUNK = _T // _NW


def _router_body(x_ref, gw_ref, eid_ref, wgt_ref, rank_ref, cnt_ref,
                 pstart_ref, te_ref, valid_ref, gx_ref):
    i = pl.program_id(0)
    x = x_ref[...]
    logits = lax.dot_general(x, gw_ref[...], (((1,), (1,)), ((), ())),
                             preferred_element_type=jnp.float32)
    lmax = jnp.max(logits, axis=1, keepdims=True)
    ex = jnp.exp(logits - lmax)
    wgt = 1.0 / jnp.sum(ex, axis=1, keepdims=True)
    eid = jnp.argmax(logits, axis=1).astype(jnp.int32)
    col = lax.broadcasted_iota(jnp.int32, (_BT, _E), 1)
    onehot = (col == eid[:, None]).astype(jnp.float32)
    r = lax.broadcasted_iota(jnp.int32, (_BT, _BT), 0)
    c = lax.broadcasted_iota(jnp.int32, (_BT, _BT), 1)
    tril = (c < r).astype(jnp.float32)
    prior = lax.dot_general(tril, onehot, (((1,), (0,)), ((), ())),
                            preferred_element_type=jnp.float32)

    @pl.when(i == 0)
    def _():
        cnt_ref[...] = jnp.zeros_like(cnt_ref)

    running = cnt_ref[...]
    rank = jnp.sum((prior + running) * onehot, axis=1, keepdims=True)
    cnt = running + jnp.sum(onehot, axis=0, keepdims=True)
    cnt_ref[...] = cnt
    eid_ref[...] = eid[:, None]
    wgt_ref[...] = wgt
    rank_ref[...] = rank.astype(jnp.int32)

    @pl.when(i == _NT - 1)
    def _():
        pc = jnp.ceil(cnt * (1.0 / _BB)) * _BB
        er = lax.broadcasted_iota(jnp.int32, (_E, _E), 0)
        ec = lax.broadcasted_iota(jnp.int32, (_E, _E), 1)
        m = (er <= ec).astype(jnp.float32)
        csum = lax.dot_general(pc, m, (((1,), (0,)), ((), ())),
                               preferred_element_type=jnp.float32)
        pstart_ref[...] = (csum - pc).astype(jnp.int32)

        total = jnp.max(csum)
        gb = (lax.broadcasted_iota(jnp.int32, (_G, 1), 0) * _BB
              ).astype(jnp.float32)
        csum_b = jnp.broadcast_to(csum, (_G, _E))
        te = jnp.sum((csum_b <= gb).astype(jnp.float32), axis=1, keepdims=True)
        last_used = jnp.sum((csum < total).astype(jnp.float32))
        te_ref[...] = jnp.minimum(te, last_used).astype(jnp.int32)
        valid_ref[...] = (gb < total).astype(jnp.int32)
        gi = lax.broadcasted_iota(jnp.int32, (_G, 1), 0).astype(jnp.float32)
        gx_ref[...] = jnp.minimum(gi, total * (1.0 / _BB) - 1.0
                                  ).astype(jnp.int32)


_NCH = 4
_CH = _CHUNK // _NCH


def _sc_scatter_body(x_hbm, wgt_hbm, eid_hbm, rank_hbm, ps_hbm,
                     xs_hbm, ww_hbm, pos_hbm,
                     rows_v, w_v, eid_v, rank_v, base_v, i0, i1, i2, i3,
                     sem_x, sem_m, sem_r, sem_w, sem_p):
    wid = lax.axis_index("s") * 2 + lax.axis_index("c")
    base = wid * _CHUNK
    idxs = (i0, i1, i2, i3)
    cp_x = [pltpu.async_copy(x_hbm.at[pl.ds(base + k * _CH, _CH)],
                             rows_v.at[pl.ds(k * _CH, _CH)], sem_x)
            for k in range(_NCH)]
    cp_e = pltpu.async_copy(eid_hbm.at[pl.ds(base, _CHUNK)], eid_v, sem_m)
    cp_k = pltpu.async_copy(rank_hbm.at[pl.ds(base, _CHUNK)], rank_v, sem_m)
    cp_w = pltpu.async_copy(wgt_hbm.at[pl.ds(base, _CHUNK)], w_v, sem_w)
    cp_e.wait()
    cp_g = pltpu.async_copy(ps_hbm.at[eid_v], base_v, sem_m)
    cp_k.wait()
    cp_g.wait()
    for k in range(_NCH):
        for j in range(_CH // 16):
            sl = pl.ds(k * _CH + j * 16, 16)
            idxs[k][pl.ds(j * 16, 16)] = base_v[sl] + rank_v[sl]
    pend = [pltpu.async_copy(idxs[k], pos_hbm.at[pl.ds(base + k * _CH, _CH)],
                             sem_p)
            for k in range(_NCH)]
    cp_w.wait()
    for k in range(_NCH):
        pend.append(pltpu.async_copy(w_v.at[pl.ds(k * _CH, _CH)],
                                     ww_hbm.at[idxs[k]], sem_w))
        cp_x[k].wait()
        pend.append(pltpu.async_copy(rows_v.at[pl.ds(k * _CH, _CH)],
                                     xs_hbm.at[idxs[k]], sem_r))
    for cp in pend:
        cp.wait()


def _sc_gather_body(ys_hbm, pos_hbm, out_hbm, idx_v, rows_v, sem):
    wid = lax.axis_index("s") * 2 + lax.axis_index("c")
    base = wid * _CHUNK
    pltpu.sync_copy(pos_hbm.at[pl.ds(base, _CHUNK)], idx_v)
    pltpu.async_copy(ys_hbm.at[idx_v], rows_v, sem).wait()
    pltpu.sync_copy(rows_v, out_hbm.at[pl.ds(base, _CHUNK)])


def _moe_body(te_ref, valid_ref, gx_ref, xs_ref, wsg_ref, wsu_ref,
              w2a_ref, w2b_ref, ww_ref, out_ref):
    g = pl.program_id(0)

    @pl.when(valid_ref[g] != 0)
    def _():
        x = xs_ref[...]
        gate = lax.dot_general(x, wsg_ref[0, 0], (((1,), (1,)), ((), ())),
                               preferred_element_type=jnp.float32)
        up = lax.dot_general(x, wsu_ref[0, 0], (((1,), (1,)), ((), ())),
                             preferred_element_type=jnp.float32)
        h = (gate * jax.nn.sigmoid(gate) * up) * ww_ref[...]
        ya = lax.dot_general(h, w2a_ref[0], (((1,), (1,)), ((), ())),
                             preferred_element_type=jnp.float32)
        yb = lax.dot_general(h, w2b_ref[0], (((1,), (1,)), ((), ())),
                             preferred_element_type=jnp.float32)
        out_ref[:, : _H // 2] = ya
        out_ref[:, _H // 2:] = yb


def _router_call(x, gate_w):
    return pl.pallas_call(
        _router_body,
        grid=(_NT,),
        in_specs=[
            pl.BlockSpec((_BT, _H), lambda i: (i, 0)),
            pl.BlockSpec((_E, _H), lambda i: (0, 0)),
        ],
        out_specs=[
            pl.BlockSpec((_BT, 1), lambda i: (i, 0)),
            pl.BlockSpec((_BT, 1), lambda i: (i, 0)),
            pl.BlockSpec((_BT, 1), lambda i: (i, 0)),
            pl.BlockSpec((1, _E), lambda i: (0, 0)),
            pl.BlockSpec((1, _E), lambda i: (0, 0)),
            pl.BlockSpec((_G, 1), lambda i: (0, 0)),
            pl.BlockSpec((_G, 1), lambda i: (0, 0)),
            pl.BlockSpec((_G, 1), lambda i: (0, 0)),
        ],
        out_shape=[
            jax.ShapeDtypeStruct((_T, 1), jnp.int32),
            jax.ShapeDtypeStruct((_T, 1), jnp.float32),
            jax.ShapeDtypeStruct((_T, 1), jnp.int32),
            jax.ShapeDtypeStruct((1, _E), jnp.float32),
            jax.ShapeDtypeStruct((1, _E), jnp.int32),
            jax.ShapeDtypeStruct((_G, 1), jnp.int32),
            jax.ShapeDtypeStruct((_G, 1), jnp.int32),
            jax.ShapeDtypeStruct((_G, 1), jnp.int32),
        ],
    )(x, gate_w)


def _scatter_call(x, wgt, eid, rank, pstart):
    f = pl.kernel(
        _sc_scatter_body,
        out_type=[
            jax.ShapeDtypeStruct((_P, _H), jnp.float32),
            jax.ShapeDtypeStruct((_P,), jnp.float32),
            jax.ShapeDtypeStruct((_T,), jnp.int32),
        ],
        mesh=plsc.VectorSubcoreMesh(core_axis_name="c", subcore_axis_name="s"),
        scratch_types=[
            pltpu.VMEM((_CHUNK, _H), jnp.float32),
            pltpu.VMEM((_CHUNK,), jnp.float32),
            pltpu.VMEM((_CHUNK,), jnp.int32),
            pltpu.VMEM((_CHUNK,), jnp.int32),
            pltpu.VMEM((_CHUNK,), jnp.int32),
            pltpu.VMEM((_CH,), jnp.int32),
            pltpu.VMEM((_CH,), jnp.int32),
            pltpu.VMEM((_CH,), jnp.int32),
            pltpu.VMEM((_CH,), jnp.int32),
            pltpu.SemaphoreType.DMA,
            pltpu.SemaphoreType.DMA,
            pltpu.SemaphoreType.DMA,
            pltpu.SemaphoreType.DMA,
            pltpu.SemaphoreType.DMA,
        ],
    )
    return f(x, wgt, eid, rank, pstart)


def _gather_call(ys, pos):
    f = pl.kernel(
        _sc_gather_body,
        out_type=jax.ShapeDtypeStruct((_T, _H), jnp.float32),
        mesh=plsc.VectorSubcoreMesh(core_axis_name="c", subcore_axis_name="s"),
        scratch_types=[
            pltpu.VMEM((_CHUNK,), jnp.int32),
            pltpu.VMEM((_CHUNK, _H), jnp.float32),
            pltpu.SemaphoreType.DMA,
        ],
    )
    return f(ys, pos)


def _moe_call(te, valid, gx, xs, ws, w2s, ww):
    ws4 = ws.reshape(_E, 2, _FF, _H)
    grid_spec = pltpu.PrefetchScalarGridSpec(
        num_scalar_prefetch=3,
        grid=(_G,),
        in_specs=[
            pl.BlockSpec((_BB, _H), lambda g, te, v, gx: (gx[g], 0)),
            pl.BlockSpec((1, 1, _FF, _H),
                         lambda g, te, v, gx: (te[g], 0, 0, 0)),
            pl.BlockSpec((1, 1, _FF, _H),
                         lambda g, te, v, gx: (te[g], 1, 0, 0)),
            pl.BlockSpec((1, _H // 2, _FF),
                         lambda g, te, v, gx: (te[g], 0, 0)),
            pl.BlockSpec((1, _H // 2, _FF),
                         lambda g, te, v, gx: (te[g], 1, 0)),
            pl.BlockSpec((_BB, 1), lambda g, te, v, gx: (gx[g], 0)),
        ],
        out_specs=pl.BlockSpec((_BB, _H), lambda g, te, v, gx: (gx[g], 0)),
    )
    return pl.pallas_call(
        _moe_body,
        grid_spec=grid_spec,
        out_shape=jax.ShapeDtypeStruct((_P, _H), jnp.float32),
        compiler_params=pltpu.CompilerParams(
            vmem_limit_bytes=100 * 1024 * 1024,
        ),
    )(te, valid, gx, xs, ws4, ws4, w2s, w2s, ww)


def kernel(hidden_states, gate_w, ws, w2s):
    x = hidden_states
    eid, wgt, rank, _, pstart, te, valid, gx = _router_call(x, gate_w)
    xs, ww, pos = _scatter_call(x, wgt.reshape(_T), eid.reshape(_T),
                                rank.reshape(_T), pstart.reshape(_E))
    ys = _moe_call(te.reshape(_G), valid.reshape(_G), gx.reshape(_G),
                   xs, ws, w2s, ww.reshape(_P, 1))
    return _gather_call(ys, pos)


# speedup vs baseline: 1.7957x; 1.0004x over previous
import functools

import jax
import jax.numpy as jnp
from jax import lax
from jax.experimental import pallas as pl
from jax.experimental.pallas import tpu as pltpu
from jax.experimental.pallas import tpu_sc as plsc

_H = 768
_FF = 1536
_E = 64
_T = 4096
_BT = 1024
_NT = _T // _BT
_BB = 128
_P = _T + _E * _BB
_G = _P // _BB
_NW = 32
_CHUNK = _T // _NW


def _router_body(x_ref, gw_ref, eid_ref, wgt_ref, rank_ref, cnt_ref,
                 pstart_ref, te_ref, valid_ref, gx_ref):
    i = pl.program_id(0)
    x = x_ref[...]
    logits = lax.dot_general(x, gw_ref[...], (((1,), (1,)), ((), ())),
                             preferred_element_type=jnp.float32)
    lmax = jnp.max(logits, axis=1, keepdims=True)
    ex = jnp.exp(logits - lmax)
    wgt = 1.0 / jnp.sum(ex, axis=1, keepdims=True)
    eid = jnp.argmax(logits, axis=1).astype(jnp.int32)
    col = lax.broadcasted_iota(jnp.int32, (_BT, _E), 1)
    onehot = (col == eid[:, None]).astype(jnp.float32)
    r = lax.broadcasted_iota(jnp.int32, (_BT, _BT), 0)
    c = lax.broadcasted_iota(jnp.int32, (_BT, _BT), 1)
    tril = (c < r).astype(jnp.float32)
    prior = lax.dot_general(tril, onehot, (((1,), (0,)), ((), ())),
                            preferred_element_type=jnp.float32)

    @pl.when(i == 0)
    def _():
        cnt_ref[...] = jnp.zeros_like(cnt_ref)

    running = cnt_ref[...]
    rank = jnp.sum((prior + running) * onehot, axis=1, keepdims=True)
    cnt = running + jnp.sum(onehot, axis=0, keepdims=True)
    cnt_ref[...] = cnt
    eid_ref[...] = eid[:, None]
    wgt_ref[...] = wgt
    rank_ref[...] = rank.astype(jnp.int32)

    @pl.when(i == _NT - 1)
    def _():
        pc = jnp.ceil(cnt * (1.0 / _BB)) * _BB
        er = lax.broadcasted_iota(jnp.int32, (_E, _E), 0)
        ec = lax.broadcasted_iota(jnp.int32, (_E, _E), 1)
        m = (er <= ec).astype(jnp.float32)
        csum = lax.dot_general(pc, m, (((1,), (0,)), ((), ())),
                               preferred_element_type=jnp.float32)
        pstart_ref[...] = (csum - pc).astype(jnp.int32)

        total = jnp.max(csum)
        gb = (lax.broadcasted_iota(jnp.int32, (_G, 1), 0) * _BB
              ).astype(jnp.float32)
        csum_b = jnp.broadcast_to(csum, (_G, _E))
        te = jnp.sum((csum_b <= gb).astype(jnp.float32), axis=1, keepdims=True)
        last_used = jnp.sum((csum < total).astype(jnp.float32))
        te_ref[...] = jnp.minimum(te, last_used).astype(jnp.int32)
        valid_ref[...] = (gb < total).astype(jnp.int32)
        gi = lax.broadcasted_iota(jnp.int32, (_G, 1), 0).astype(jnp.float32)
        gx_ref[...] = jnp.minimum(gi, total * (1.0 / _BB) - 1.0
                                  ).astype(jnp.int32)


_NCH = 8
_CH = _CHUNK // _NCH


def _sc_scatter_body(x_hbm, wgt_hbm, eid_hbm, rank_hbm, ps_hbm,
                     xs_hbm, ww_hbm, pos_hbm,
                     rows_v, w_v, eid_v, rank_v, base_v,
                     i0, i1, i2, i3, i4, i5, i6, i7,
                     sem_x, sem_m, sem_r, sem_w, sem_p):
    wid = lax.axis_index("s") * 2 + lax.axis_index("c")
    base = wid * _CHUNK
    idxs = (i0, i1, i2, i3, i4, i5, i6, i7)
    cp_x = [pltpu.async_copy(x_hbm.at[pl.ds(base + k * _CH, _CH)],
                             rows_v.at[pl.ds(k * _CH, _CH)], sem_x)
            for k in range(_NCH)]
    cp_e = pltpu.async_copy(eid_hbm.at[pl.ds(base, _CHUNK)], eid_v, sem_m)
    cp_k = pltpu.async_copy(rank_hbm.at[pl.ds(base, _CHUNK)], rank_v, sem_m)
    cp_w = pltpu.async_copy(wgt_hbm.at[pl.ds(base, _CHUNK)], w_v, sem_w)
    cp_e.wait()
    cp_g = pltpu.async_copy(ps_hbm.at[eid_v], base_v, sem_m)
    cp_k.wait()
    cp_g.wait()
    for k in range(_NCH):
        for j in range(_CH // 16):
            sl = pl.ds(k * _CH + j * 16, 16)
            idxs[k][pl.ds(j * 16, 16)] = base_v[sl] + rank_v[sl]
    pend = [pltpu.async_copy(idxs[k], pos_hbm.at[pl.ds(base + k * _CH, _CH)],
                             sem_p)
            for k in range(_NCH)]
    cp_w.wait()
    for k in range(_NCH):
        pend.append(pltpu.async_copy(w_v.at[pl.ds(k * _CH, _CH)],
                                     ww_hbm.at[idxs[k]], sem_w))
        cp_x[k].wait()
        pend.append(pltpu.async_copy(rows_v.at[pl.ds(k * _CH, _CH)],
                                     xs_hbm.at[idxs[k]], sem_r))
    for cp in pend:
        cp.wait()


def _sc_gather_body(ys_hbm, pos_hbm, out_hbm, idx_v, rows_v, sem):
    wid = lax.axis_index("s") * 2 + lax.axis_index("c")
    base = wid * _CHUNK
    pltpu.sync_copy(pos_hbm.at[pl.ds(base, _CHUNK)], idx_v)
    pltpu.async_copy(ys_hbm.at[idx_v], rows_v, sem).wait()
    pltpu.sync_copy(rows_v, out_hbm.at[pl.ds(base, _CHUNK)])


def _moe_body(te_ref, valid_ref, gx_ref, xs_ref, wsg_ref, wsu_ref,
              w2a_ref, w2b_ref, ww_ref, out_ref):
    g = pl.program_id(0)

    @pl.when(valid_ref[g] != 0)
    def _():
        x = xs_ref[...]
        gate = lax.dot_general(x, wsg_ref[0, 0], (((1,), (1,)), ((), ())),
                               preferred_element_type=jnp.float32)
        up = lax.dot_general(x, wsu_ref[0, 0], (((1,), (1,)), ((), ())),
                             preferred_element_type=jnp.float32)
        h = (gate * jax.nn.sigmoid(gate) * up) * ww_ref[...]
        ya = lax.dot_general(h, w2a_ref[0], (((1,), (1,)), ((), ())),
                             preferred_element_type=jnp.float32)
        yb = lax.dot_general(h, w2b_ref[0], (((1,), (1,)), ((), ())),
                             preferred_element_type=jnp.float32)
        out_ref[:, : _H // 2] = ya
        out_ref[:, _H // 2:] = yb


def _router_call(x, gate_w):
    return pl.pallas_call(
        _router_body,
        grid=(_NT,),
        in_specs=[
            pl.BlockSpec((_BT, _H), lambda i: (i, 0)),
            pl.BlockSpec((_E, _H), lambda i: (0, 0)),
        ],
        out_specs=[
            pl.BlockSpec((_BT, 1), lambda i: (i, 0)),
            pl.BlockSpec((_BT, 1), lambda i: (i, 0)),
            pl.BlockSpec((_BT, 1), lambda i: (i, 0)),
            pl.BlockSpec((1, _E), lambda i: (0, 0)),
            pl.BlockSpec((1, _E), lambda i: (0, 0)),
            pl.BlockSpec((_G, 1), lambda i: (0, 0)),
            pl.BlockSpec((_G, 1), lambda i: (0, 0)),
            pl.BlockSpec((_G, 1), lambda i: (0, 0)),
        ],
        out_shape=[
            jax.ShapeDtypeStruct((_T, 1), jnp.int32),
            jax.ShapeDtypeStruct((_T, 1), jnp.float32),
            jax.ShapeDtypeStruct((_T, 1), jnp.int32),
            jax.ShapeDtypeStruct((1, _E), jnp.float32),
            jax.ShapeDtypeStruct((1, _E), jnp.int32),
            jax.ShapeDtypeStruct((_G, 1), jnp.int32),
            jax.ShapeDtypeStruct((_G, 1), jnp.int32),
            jax.ShapeDtypeStruct((_G, 1), jnp.int32),
        ],
    )(x, gate_w)


def _scatter_call(x, wgt, eid, rank, pstart):
    f = pl.kernel(
        _sc_scatter_body,
        out_type=[
            jax.ShapeDtypeStruct((_P, _H), jnp.float32),
            jax.ShapeDtypeStruct((_P,), jnp.float32),
            jax.ShapeDtypeStruct((_T,), jnp.int32),
        ],
        mesh=plsc.VectorSubcoreMesh(core_axis_name="c", subcore_axis_name="s"),
        scratch_types=[
            pltpu.VMEM((_CHUNK, _H), jnp.float32),
            pltpu.VMEM((_CHUNK,), jnp.float32),
            pltpu.VMEM((_CHUNK,), jnp.int32),
            pltpu.VMEM((_CHUNK,), jnp.int32),
            pltpu.VMEM((_CHUNK,), jnp.int32),
            pltpu.VMEM((_CH,), jnp.int32),
            pltpu.VMEM((_CH,), jnp.int32),
            pltpu.VMEM((_CH,), jnp.int32),
            pltpu.VMEM((_CH,), jnp.int32),
            pltpu.VMEM((_CH,), jnp.int32),
            pltpu.VMEM((_CH,), jnp.int32),
            pltpu.VMEM((_CH,), jnp.int32),
            pltpu.VMEM((_CH,), jnp.int32),
            pltpu.SemaphoreType.DMA,
            pltpu.SemaphoreType.DMA,
            pltpu.SemaphoreType.DMA,
            pltpu.SemaphoreType.DMA,
            pltpu.SemaphoreType.DMA,
        ],
    )
    return f(x, wgt, eid, rank, pstart)


def _gather_call(ys, pos):
    f = pl.kernel(
        _sc_gather_body,
        out_type=jax.ShapeDtypeStruct((_T, _H), jnp.float32),
        mesh=plsc.VectorSubcoreMesh(core_axis_name="c", subcore_axis_name="s"),
        scratch_types=[
            pltpu.VMEM((_CHUNK,), jnp.int32),
            pltpu.VMEM((_CHUNK, _H), jnp.float32),
            pltpu.SemaphoreType.DMA,
        ],
    )
    return f(ys, pos)


def _moe_call(te, valid, gx, xs, ws, w2s, ww):
    ws4 = ws.reshape(_E, 2, _FF, _H)
    grid_spec = pltpu.PrefetchScalarGridSpec(
        num_scalar_prefetch=3,
        grid=(_G,),
        in_specs=[
            pl.BlockSpec((_BB, _H), lambda g, te, v, gx: (gx[g], 0)),
            pl.BlockSpec((1, 1, _FF, _H),
                         lambda g, te, v, gx: (te[g], 0, 0, 0)),
            pl.BlockSpec((1, 1, _FF, _H),
                         lambda g, te, v, gx: (te[g], 1, 0, 0)),
            pl.BlockSpec((1, _H // 2, _FF),
                         lambda g, te, v, gx: (te[g], 0, 0)),
            pl.BlockSpec((1, _H // 2, _FF),
                         lambda g, te, v, gx: (te[g], 1, 0)),
            pl.BlockSpec((_BB, 1), lambda g, te, v, gx: (gx[g], 0)),
        ],
        out_specs=pl.BlockSpec((_BB, _H), lambda g, te, v, gx: (gx[g], 0)),
    )
    return pl.pallas_call(
        _moe_body,
        grid_spec=grid_spec,
        out_shape=jax.ShapeDtypeStruct((_P, _H), jnp.float32),
        compiler_params=pltpu.CompilerParams(
            vmem_limit_bytes=100 * 1024 * 1024,
        ),
    )(te, valid, gx, xs, ws4, ws4, w2s, w2s, ww)


def kernel(hidden_states, gate_w, ws, w2s):
    x = hidden_states
    eid, wgt, rank, _, pstart, te, valid, gx = _router_call(x, gate_w)
    xs, ww, pos = _scatter_call(x, wgt.reshape(_T), eid.reshape(_T),
                                rank.reshape(_T), pstart.reshape(_E))
    ys = _moe_call(te.reshape(_G), valid.reshape(_G), gx.reshape(_G),
                   xs, ws, w2s, ww.reshape(_P, 1))
    return _gather_call(ys, pos)


# speedup vs baseline: 1.7995x; 1.0021x over previous
import jax
import jax.numpy as jnp
from jax import lax
from jax.experimental import pallas as pl
from jax.experimental.pallas import tpu as pltpu
from jax.experimental.pallas import tpu_sc as plsc

_H = 768
_FF = 1536
_E = 64
_T = 4096
_BT = 1024
_NT = _T // _BT
_BB = 128
_P = _T + _E * _BB
_G = _P // _BB
_NW = 32
_CHUNK = _T // _NW


def _router_body(x_ref, gw_ref, eid_ref, wgt_ref, rank_ref, cnt_ref,
                 pstart_ref, te_ref, valid_ref, gx_ref):
    i = pl.program_id(0)
    x = x_ref[...]
    logits = lax.dot_general(x, gw_ref[...], (((1,), (1,)), ((), ())),
                             preferred_element_type=jnp.float32)
    lmax = jnp.max(logits, axis=1, keepdims=True)
    ex = jnp.exp(logits - lmax)
    wgt = 1.0 / jnp.sum(ex, axis=1, keepdims=True)
    eid = jnp.argmax(logits, axis=1).astype(jnp.int32)
    col = lax.broadcasted_iota(jnp.int32, (_BT, _E), 1)
    onehot = (col == eid[:, None]).astype(jnp.float32)
    r = lax.broadcasted_iota(jnp.int32, (_BT, _BT), 0)
    c = lax.broadcasted_iota(jnp.int32, (_BT, _BT), 1)
    tril = (c < r).astype(jnp.float32)
    prior = lax.dot_general(tril, onehot, (((1,), (0,)), ((), ())),
                            preferred_element_type=jnp.float32)

    @pl.when(i == 0)
    def _():
        cnt_ref[...] = jnp.zeros_like(cnt_ref)

    running = cnt_ref[...]
    rank = jnp.sum((prior + running) * onehot, axis=1, keepdims=True)
    cnt = running + jnp.sum(onehot, axis=0, keepdims=True)
    cnt_ref[...] = cnt
    eid_ref[...] = eid[:, None]
    wgt_ref[...] = wgt
    rank_ref[...] = rank.astype(jnp.int32)

    @pl.when(i == _NT - 1)
    def _():
        pc = jnp.ceil(cnt * (1.0 / _BB)) * _BB
        er = lax.broadcasted_iota(jnp.int32, (_E, _E), 0)
        ec = lax.broadcasted_iota(jnp.int32, (_E, _E), 1)
        m = (er <= ec).astype(jnp.float32)
        csum = lax.dot_general(pc, m, (((1,), (0,)), ((), ())),
                               preferred_element_type=jnp.float32)
        pstart_ref[...] = (csum - pc).astype(jnp.int32)

        total = jnp.max(csum)
        gb = (lax.broadcasted_iota(jnp.int32, (_G, 1), 0) * _BB
              ).astype(jnp.float32)
        csum_b = jnp.broadcast_to(csum, (_G, _E))
        te = jnp.sum((csum_b <= gb).astype(jnp.float32), axis=1, keepdims=True)
        last_used = jnp.sum((csum < total).astype(jnp.float32))
        te_ref[...] = jnp.minimum(te, last_used).astype(jnp.int32)
        valid_ref[...] = (gb < total).astype(jnp.int32)
        gi = lax.broadcasted_iota(jnp.int32, (_G, 1), 0).astype(jnp.float32)
        gx_ref[...] = jnp.minimum(gi, total * (1.0 / _BB) - 1.0
                                  ).astype(jnp.int32)


_NCH = 8
_CH = _CHUNK // _NCH


def _sc_scatter_body(x_hbm, wgt_hbm, eid_hbm, rank_hbm, ps_hbm,
                     xs_hbm, ww_hbm, pos_hbm,
                     rows_v, w_v, eid_v, rank_v, base_v,
                     i0, i1, i2, i3, i4, i5, i6, i7,
                     sem_x, sem_m, sem_r, sem_w, sem_p):
    wid = lax.axis_index("s") * 2 + lax.axis_index("c")
    base = wid * _CHUNK
    idxs = (i0, i1, i2, i3, i4, i5, i6, i7)
    cp_x = [pltpu.async_copy(x_hbm.at[pl.ds(base + k * _CH, _CH)],
                             rows_v.at[pl.ds(k * _CH, _CH)], sem_x)
            for k in range(_NCH)]
    cp_e = pltpu.async_copy(eid_hbm.at[pl.ds(base, _CHUNK)], eid_v, sem_m)
    cp_k = pltpu.async_copy(rank_hbm.at[pl.ds(base, _CHUNK)], rank_v, sem_m)
    cp_w = pltpu.async_copy(wgt_hbm.at[pl.ds(base, _CHUNK)], w_v, sem_w)
    cp_e.wait()
    cp_g = pltpu.async_copy(ps_hbm.at[eid_v], base_v, sem_m)
    cp_k.wait()
    cp_g.wait()
    for k in range(_NCH):
        for j in range(_CH // 16):
            sl = pl.ds(k * _CH + j * 16, 16)
            idxs[k][pl.ds(j * 16, 16)] = base_v[sl] + rank_v[sl]
    pend = [pltpu.async_copy(idxs[k], pos_hbm.at[pl.ds(base + k * _CH, _CH)],
                             sem_p)
            for k in range(_NCH)]
    cp_w.wait()
    for k in range(_NCH):
        pend.append(pltpu.async_copy(w_v.at[pl.ds(k * _CH, _CH)],
                                     ww_hbm.at[idxs[k]], sem_w))
        cp_x[k].wait()
        pend.append(pltpu.async_copy(rows_v.at[pl.ds(k * _CH, _CH)],
                                     xs_hbm.at[idxs[k]], sem_r))
    for cp in pend:
        cp.wait()


def _sc_gather_body(ys_hbm, pos_hbm, out_hbm, idx_v, rows_v, sem):
    wid = lax.axis_index("s") * 2 + lax.axis_index("c")
    base = wid * _CHUNK
    pltpu.sync_copy(pos_hbm.at[pl.ds(base, _CHUNK)], idx_v)
    pltpu.async_copy(ys_hbm.at[idx_v], rows_v, sem).wait()
    pltpu.sync_copy(rows_v, out_hbm.at[pl.ds(base, _CHUNK)])


def _moe_body(te_ref, valid_ref, gx_ref, xs_ref, wsg_ref, wsu_ref,
              w2a_ref, w2b_ref, ww_ref, out_ref):
    g = pl.program_id(0)

    @pl.when(valid_ref[g] != 0)
    def _():
        x = xs_ref[...]
        gate = lax.dot_general(x, wsg_ref[0, 0], (((1,), (1,)), ((), ())),
                               preferred_element_type=jnp.float32)
        up = lax.dot_general(x, wsu_ref[0, 0], (((1,), (1,)), ((), ())),
                             preferred_element_type=jnp.float32)
        h = (gate * jax.nn.sigmoid(gate) * up) * ww_ref[...]
        ya = lax.dot_general(h, w2a_ref[0], (((1,), (1,)), ((), ())),
                             preferred_element_type=jnp.float32)
        yb = lax.dot_general(h, w2b_ref[0], (((1,), (1,)), ((), ())),
                             preferred_element_type=jnp.float32)
        out_ref[:, : _H // 2] = ya
        out_ref[:, _H // 2:] = yb


def _router_call(x, gate_w):
    return pl.pallas_call(
        _router_body,
        grid=(_NT,),
        in_specs=[
            pl.BlockSpec((_BT, _H), lambda i: (i, 0)),
            pl.BlockSpec((_E, _H), lambda i: (0, 0)),
        ],
        out_specs=[
            pl.BlockSpec((_BT, 1), lambda i: (i, 0)),
            pl.BlockSpec((_BT, 1), lambda i: (i, 0)),
            pl.BlockSpec((_BT, 1), lambda i: (i, 0)),
            pl.BlockSpec((1, _E), lambda i: (0, 0)),
            pl.BlockSpec((1, _E), lambda i: (0, 0)),
            pl.BlockSpec((_G, 1), lambda i: (0, 0)),
            pl.BlockSpec((_G, 1), lambda i: (0, 0)),
            pl.BlockSpec((_G, 1), lambda i: (0, 0)),
        ],
        out_shape=[
            jax.ShapeDtypeStruct((_T, 1), jnp.int32),
            jax.ShapeDtypeStruct((_T, 1), jnp.float32),
            jax.ShapeDtypeStruct((_T, 1), jnp.int32),
            jax.ShapeDtypeStruct((1, _E), jnp.float32),
            jax.ShapeDtypeStruct((1, _E), jnp.int32),
            jax.ShapeDtypeStruct((_G, 1), jnp.int32),
            jax.ShapeDtypeStruct((_G, 1), jnp.int32),
            jax.ShapeDtypeStruct((_G, 1), jnp.int32),
        ],
    )(x, gate_w)


def _scatter_call(x, wgt, eid, rank, pstart):
    f = pl.kernel(
        _sc_scatter_body,
        out_type=[
            jax.ShapeDtypeStruct((_P, _H), jnp.float32),
            jax.ShapeDtypeStruct((_P,), jnp.float32),
            jax.ShapeDtypeStruct((_T,), jnp.int32),
        ],
        mesh=plsc.VectorSubcoreMesh(core_axis_name="c", subcore_axis_name="s"),
        scratch_types=[
            pltpu.VMEM((_CHUNK, _H), jnp.float32),
            pltpu.VMEM((_CHUNK,), jnp.float32),
            pltpu.VMEM((_CHUNK,), jnp.int32),
            pltpu.VMEM((_CHUNK,), jnp.int32),
            pltpu.VMEM((_CHUNK,), jnp.int32),
            pltpu.VMEM((_CH,), jnp.int32),
            pltpu.VMEM((_CH,), jnp.int32),
            pltpu.VMEM((_CH,), jnp.int32),
            pltpu.VMEM((_CH,), jnp.int32),
            pltpu.VMEM((_CH,), jnp.int32),
            pltpu.VMEM((_CH,), jnp.int32),
            pltpu.VMEM((_CH,), jnp.int32),
            pltpu.VMEM((_CH,), jnp.int32),
            pltpu.SemaphoreType.DMA,
            pltpu.SemaphoreType.DMA,
            pltpu.SemaphoreType.DMA,
            pltpu.SemaphoreType.DMA,
            pltpu.SemaphoreType.DMA,
        ],
    )
    return f(x, wgt, eid, rank, pstart)


def _gather_call(ys, pos):
    f = pl.kernel(
        _sc_gather_body,
        out_type=jax.ShapeDtypeStruct((_T, _H), jnp.float32),
        mesh=plsc.VectorSubcoreMesh(core_axis_name="c", subcore_axis_name="s"),
        scratch_types=[
            pltpu.VMEM((_CHUNK,), jnp.int32),
            pltpu.VMEM((_CHUNK, _H), jnp.float32),
            pltpu.SemaphoreType.DMA,
        ],
    )
    return f(ys, pos)


def _moe_call(te, valid, gx, xs, ws, w2s, ww):
    ws4 = ws.reshape(_E, 2, _FF, _H)
    grid_spec = pltpu.PrefetchScalarGridSpec(
        num_scalar_prefetch=3,
        grid=(_G,),
        in_specs=[
            pl.BlockSpec((_BB, _H), lambda g, te, v, gx: (gx[g], 0)),
            pl.BlockSpec((1, 1, _FF, _H),
                         lambda g, te, v, gx: (te[g], 0, 0, 0)),
            pl.BlockSpec((1, 1, _FF, _H),
                         lambda g, te, v, gx: (te[g], 1, 0, 0)),
            pl.BlockSpec((1, _H // 2, _FF),
                         lambda g, te, v, gx: (te[g], 0, 0)),
            pl.BlockSpec((1, _H // 2, _FF),
                         lambda g, te, v, gx: (te[g], 1, 0)),
            pl.BlockSpec((_BB, 1), lambda g, te, v, gx: (gx[g], 0)),
        ],
        out_specs=pl.BlockSpec((_BB, _H), lambda g, te, v, gx: (gx[g], 0)),
    )
    return pl.pallas_call(
        _moe_body,
        grid_spec=grid_spec,
        out_shape=jax.ShapeDtypeStruct((_P, _H), jnp.float32),
        compiler_params=pltpu.CompilerParams(
            vmem_limit_bytes=100 * 1024 * 1024,
        ),
    )(te, valid, gx, xs, ws4, ws4, w2s, w2s, ww)


def kernel(hidden_states, gate_w, ws, w2s):
    x = hidden_states
    eid, wgt, rank, _, pstart, te, valid, gx = _router_call(x, gate_w)
    xs, ww, pos = _scatter_call(x, wgt.reshape(_T), eid.reshape(_T),
                                rank.reshape(_T), pstart.reshape(_E))
    ys = _moe_call(te.reshape(_G), valid.reshape(_G), gx.reshape(_G),
                   xs, ws, w2s, ww.reshape(_P, 1))
    return _gather_call(ys, pos)
